# baseline (speedup 1.0000x reference)
import functools

import jax
import jax.numpy as jnp
from jax import lax
from jax.experimental import pallas as pl
from jax.experimental.pallas import tpu as pltpu
from jax.experimental.pallas import tpu_sc as plsc

N = 10000
D = 128
E = 320000

NC = 2
NS = 16
NW = NC * NS
CPT = D // NW
ROWS_PAD = 10240
L = 3200
NCHK = E // L
NBUF = 4
NQ = NCHK // NBUF


@functools.cache
def _build_sc_agg():
    mesh = plsc.VectorSubcoreMesh(
        core_axis_name="c", subcore_axis_name="s",
        num_cores=NC, num_subcores=NS)

    @functools.partial(
        pl.kernel,
        out_type=jax.ShapeDtypeStruct((NW, CPT, ROWS_PAD), jnp.float32),
        mesh=mesh,
        scratch_types=[
            pltpu.VMEM((CPT * ROWS_PAD,), jnp.float32),
            pltpu.VMEM((CPT * ROWS_PAD,), jnp.float32),
            pltpu.VMEM((NBUF, L), jnp.int32),
            pltpu.VMEM((NBUF, L), jnp.int32),
            [pltpu.SemaphoreType.DMA] * NBUF,
            [pltpu.SemaphoreType.DMA] * NBUF,
        ],
        compiler_params=pltpu.CompilerParams(needs_layout_passes=False),
    )
    def sc_agg(ht_hbm, src_hbm, dst_hbm, zeros_hbm, out_hbm,
               h_v, agg_v, sv, dv, sems_s, sems_d):
        c = lax.axis_index("c")
        s = lax.axis_index("s")
        w = c * NS + s

        for j in range(CPT):
            pltpu.sync_copy(ht_hbm.at[w, j],
                            h_v.at[pl.ds(j * ROWS_PAD, ROWS_PAD)])
        pltpu.sync_copy(zeros_hbm, agg_v)

        for b in range(NBUF):
            pltpu.async_copy(src_hbm.at[pl.ds(b * L, L)], sv.at[b],
                             sems_s[b])
            pltpu.async_copy(dst_hbm.at[pl.ds(b * L, L)], dv.at[b],
                             sems_d[b])

        def quad(q, carry):
            for b in range(NBUF):
                ch = q * NBUF + b
                pltpu.make_async_copy(
                    src_hbm.at[pl.ds(ch * L, L)], sv.at[b], sems_s[b]).wait()
                pltpu.make_async_copy(
                    dst_hbm.at[pl.ds(ch * L, L)], dv.at[b], sems_d[b]).wait()

                @plsc.parallel_loop(0, L // 64, unroll=4)
                def _group(g):
                    for u in range(4):
                        k = (g * 4 + u) * 16
                        srcg = sv[b, pl.ds(k, 16)]
                        dstg = dv[b, pl.ds(k, 16)]
                        for j in range(CPT):
                            off = jnp.int32(j * ROWS_PAD)
                            v = plsc.load_gather(h_v, [srcg + off])
                            plsc.addupdate_scatter(agg_v, [dstg + off], v)

                cn = lax.min(ch + NBUF, NCHK - 1)
                pltpu.async_copy(src_hbm.at[pl.ds(cn * L, L)], sv.at[b],
                                 sems_s[b])
                pltpu.async_copy(dst_hbm.at[pl.ds(cn * L, L)], dv.at[b],
                                 sems_d[b])
            return carry

        lax.fori_loop(0, NQ, quad, 0)

        for b in range(NBUF):
            pltpu.make_async_copy(
                src_hbm.at[pl.ds(0, L)], sv.at[b], sems_s[b]).wait()
            pltpu.make_async_copy(
                dst_hbm.at[pl.ds(0, L)], dv.at[b], sems_d[b]).wait()

        for j in range(CPT):
            pltpu.sync_copy(agg_v.at[pl.ds(j * ROWS_PAD, ROWS_PAD)],
                            out_hbm.at[w, j])

    return sc_agg


def _sc_agg(ht, src, dst, zeros):
    return _build_sc_agg()(ht, src, dst, zeros)


BRT = 1024
NB = ROWS_PAD // BRT


def _colmask(k, x):
    col = lax.broadcasted_iota(jnp.int32, (D, BRT), 1)
    return jnp.where(col < N - k * BRT, x, 0.0)


def _layer_body(eps_ref, ht_ref, at_ref, wt_ref, b_ref,
                gi_ref, bi_ref, go_ref, bo_ref, out_ref,
                y_s, r_s, s1_s, s2_s, sc_s, sh_s):
    i = pl.program_id(0)
    k = pl.program_id(1)
    cols = pl.ds(k * BRT, BRT)

    def finalize(g_ref, be_ref):
        mu = s1_s[:, 0:1] / N
        var = s2_s[:, 0:1] / N - mu * mu
        scv = g_ref[:, 0:1] * lax.rsqrt(var + 1e-5)
        sc_s[...] = jnp.broadcast_to(scv, (D, 128))
        sh_s[...] = jnp.broadcast_to(be_ref[:, 0:1] - mu * scv, (D, 128))
        s1_s[...] = jnp.zeros_like(s1_s)
        s2_s[...] = jnp.zeros_like(s2_s)

    def accum(v):
        s1_s[...] += jnp.broadcast_to(
            jnp.sum(v, axis=1, keepdims=True), (D, 128))
        s2_s[...] += jnp.broadcast_to(
            jnp.sum(v * v, axis=1, keepdims=True), (D, 128))

    @pl.when(i == 0)
    def _():
        @pl.when(k == 0)
        def _():
            s1_s[...] = jnp.zeros_like(s1_s)
            s2_s[...] = jnp.zeros_like(s2_s)

        t = ht_ref[...] * eps_ref[0, 0] + at_ref[...]
        y = jnp.dot(wt_ref[...], t, preferred_element_type=jnp.float32)
        y = y + b_ref[:, 0:1]
        y_s[:, cols] = y
        accum(_colmask(k, y))

    @pl.when(i == 1)
    def _():
        @pl.when(k == 0)
        def _():
            finalize(gi_ref, bi_ref)

        r = jnp.maximum(y_s[:, cols] * sc_s[:, 0:1] + sh_s[:, 0:1], 0.0)
        r = _colmask(k, r)
        r_s[:, cols] = r
        accum(r)

    @pl.when(i == 2)
    def _():
        @pl.when(k == 0)
        def _():
            finalize(go_ref, bo_ref)

        r2 = jnp.maximum(r_s[:, cols] * sc_s[:, 0:1] + sh_s[:, 0:1], 0.0)
        out_ref[...] = _colmask(k, r2)


_col_spec = pl.BlockSpec((D, BRT), lambda i, k: (0, k))
_par_spec = pl.BlockSpec((D, 128), lambda i, k: (0, 0))

_layer_call = pl.pallas_call(
    _layer_body,
    grid=(3, NB),
    in_specs=[
        pl.BlockSpec(memory_space=pltpu.SMEM),
        _col_spec,
        _col_spec,
        pl.BlockSpec((D, D), lambda i, k: (0, 0)),
        _par_spec, _par_spec, _par_spec, _par_spec, _par_spec,
    ],
    out_specs=_col_spec,
    out_shape=jax.ShapeDtypeStruct((D, ROWS_PAD), jnp.float32),
    scratch_shapes=[
        pltpu.VMEM((D, ROWS_PAD), jnp.float32),
        pltpu.VMEM((D, ROWS_PAD), jnp.float32),
        pltpu.VMEM((D, 128), jnp.float32),
        pltpu.VMEM((D, 128), jnp.float32),
        pltpu.VMEM((D, 128), jnp.float32),
        pltpu.VMEM((D, 128), jnp.float32),
    ],
)


def _bcastD(v):
    return jnp.broadcast_to(v[:, None], (D, 128))


def _layer(ht, aggt, W, b, eps, gi, bi, go, bo):
    epsb = jnp.reshape(1.0 + eps, (1, 1))
    return _layer_call(epsb, ht, aggt, W.T, _bcastD(b),
                       _bcastD(gi), _bcastD(bi), _bcastD(go), _bcastD(bo))


def kernel(x, edge_index, W0, b0, W1, b1, eps0, eps1,
           g_in0, b_in0, g_out0, b_out0, g_in1, b_in1, g_out1, b_out1):
    src = edge_index[0].astype(jnp.int32)
    dst = edge_index[1].astype(jnp.int32)
    zeros = jnp.zeros((CPT * ROWS_PAD,), jnp.float32)

    xt = jnp.pad(x.T, ((0, 0), (0, ROWS_PAD - N)))
    agg0 = _sc_agg(xt.reshape(NW, CPT, ROWS_PAD), src, dst, zeros)
    h1t = _layer(xt, agg0.reshape(D, ROWS_PAD), W0, b0, eps0,
                 g_in0, b_in0, g_out0, b_out0)
    agg1 = _sc_agg(h1t.reshape(NW, CPT, ROWS_PAD), src, dst, zeros)
    h2t = _layer(h1t, agg1.reshape(D, ROWS_PAD), W1, b1, eps1,
                 g_in1, b_in1, g_out1, b_out1)
    return jnp.concatenate([h1t[:, :N].T, h2t[:, :N].T], axis=1)


# speedup vs baseline: 1.0511x; 1.0511x over previous
import functools

import jax
import jax.numpy as jnp
from jax import lax
from jax.experimental import pallas as pl
from jax.experimental.pallas import tpu as pltpu
from jax.experimental.pallas import tpu_sc as plsc

N = 10000
D = 128
E = 320000

NC = 2
NS = 16
NW = NC * NS
CPT = D // NW
ROWS_PAD = 10240
L = 6400
NCHK = E // L
NPAIR = NCHK // 2


@functools.cache
def _build_sc_agg():
    mesh = plsc.VectorSubcoreMesh(
        core_axis_name="c", subcore_axis_name="s",
        num_cores=NC, num_subcores=NS)

    @functools.partial(
        pl.kernel,
        out_type=jax.ShapeDtypeStruct((NW, CPT, ROWS_PAD), jnp.float32),
        mesh=mesh,
        scratch_types=[
            pltpu.VMEM((CPT * ROWS_PAD,), jnp.float32),
            pltpu.VMEM((CPT * ROWS_PAD,), jnp.float32),
            pltpu.VMEM((2, L), jnp.int32),
            pltpu.VMEM((2, L), jnp.int32),
            [pltpu.SemaphoreType.DMA] * 2,
            [pltpu.SemaphoreType.DMA] * 2,
        ],
        compiler_params=pltpu.CompilerParams(needs_layout_passes=False),
    )
    def sc_agg(ht_hbm, src_hbm, dst_hbm, zeros_hbm, out_hbm,
               h_v, agg_v, sv, dv, sems_s, sems_d):
        c = lax.axis_index("c")
        s = lax.axis_index("s")
        w = c * NS + s

        for j in range(CPT):
            pltpu.sync_copy(ht_hbm.at[w, j],
                            h_v.at[pl.ds(j * ROWS_PAD, ROWS_PAD)])
        pltpu.sync_copy(zeros_hbm, agg_v)

        def pair(p, carry):
            descs = []
            for b in range(2):
                ch = p * 2 + b
                descs.append(pltpu.async_copy(
                    src_hbm.at[pl.ds(ch * L, L)], sv.at[b], sems_s[b]))
                descs.append(pltpu.async_copy(
                    dst_hbm.at[pl.ds(ch * L, L)], dv.at[b], sems_d[b]))
            for b in range(2):
                descs[2 * b].wait()
                descs[2 * b + 1].wait()

                @plsc.parallel_loop(0, L // 64, unroll=4)
                def _group(g):
                    for u in range(4):
                        k = (g * 4 + u) * 16
                        srcg = sv[b, pl.ds(k, 16)]
                        dstg = dv[b, pl.ds(k, 16)]
                        for j in range(CPT):
                            off = jnp.int32(j * ROWS_PAD)
                            v = plsc.load_gather(h_v, [srcg + off])
                            plsc.addupdate_scatter(agg_v, [dstg + off], v)
            return carry

        lax.fori_loop(0, NPAIR, pair, 0)

        for j in range(CPT):
            pltpu.sync_copy(agg_v.at[pl.ds(j * ROWS_PAD, ROWS_PAD)],
                            out_hbm.at[w, j])

    return sc_agg


def _sc_agg(ht, src, dst, zeros):
    return _build_sc_agg()(ht, src, dst, zeros)


BRT = 1024
NB = ROWS_PAD // BRT


def _colmask(k, x):
    col = lax.broadcasted_iota(jnp.int32, (D, BRT), 1)
    return jnp.where(col < N - k * BRT, x, 0.0)


def _layer_body(eps_ref, ht_ref, at_ref, wt_ref, b_ref,
                gi_ref, bi_ref, go_ref, bo_ref, out_ref,
                y_s, r_s, s1_s, s2_s, sc_s, sh_s):
    i = pl.program_id(0)
    k = pl.program_id(1)
    cols = pl.ds(k * BRT, BRT)

    def finalize(g_ref, be_ref):
        mu = s1_s[:, 0:1] / N
        var = s2_s[:, 0:1] / N - mu * mu
        scv = g_ref[:, 0:1] * lax.rsqrt(var + 1e-5)
        sc_s[...] = jnp.broadcast_to(scv, (D, 128))
        sh_s[...] = jnp.broadcast_to(be_ref[:, 0:1] - mu * scv, (D, 128))
        s1_s[...] = jnp.zeros_like(s1_s)
        s2_s[...] = jnp.zeros_like(s2_s)

    def accum(v):
        s1_s[...] += jnp.broadcast_to(
            jnp.sum(v, axis=1, keepdims=True), (D, 128))
        s2_s[...] += jnp.broadcast_to(
            jnp.sum(v * v, axis=1, keepdims=True), (D, 128))

    @pl.when(i == 0)
    def _():
        @pl.when(k == 0)
        def _():
            s1_s[...] = jnp.zeros_like(s1_s)
            s2_s[...] = jnp.zeros_like(s2_s)

        t = ht_ref[...] * eps_ref[0, 0] + at_ref[...]
        y = jnp.dot(wt_ref[...], t, preferred_element_type=jnp.float32)
        y = y + b_ref[:, 0:1]
        y_s[:, cols] = y
        accum(_colmask(k, y))

    @pl.when(i == 1)
    def _():
        @pl.when(k == 0)
        def _():
            finalize(gi_ref, bi_ref)

        r = jnp.maximum(y_s[:, cols] * sc_s[:, 0:1] + sh_s[:, 0:1], 0.0)
        r = _colmask(k, r)
        r_s[:, cols] = r
        accum(r)

    @pl.when(i == 2)
    def _():
        @pl.when(k == 0)
        def _():
            finalize(go_ref, bo_ref)

        r2 = jnp.maximum(r_s[:, cols] * sc_s[:, 0:1] + sh_s[:, 0:1], 0.0)
        out_ref[...] = _colmask(k, r2)


_col_spec = pl.BlockSpec((D, BRT), lambda i, k: (0, k))
_par_spec = pl.BlockSpec((D, 128), lambda i, k: (0, 0))

_layer_call = pl.pallas_call(
    _layer_body,
    grid=(3, NB),
    in_specs=[
        pl.BlockSpec(memory_space=pltpu.SMEM),
        _col_spec,
        _col_spec,
        pl.BlockSpec((D, D), lambda i, k: (0, 0)),
        _par_spec, _par_spec, _par_spec, _par_spec, _par_spec,
    ],
    out_specs=_col_spec,
    out_shape=jax.ShapeDtypeStruct((D, ROWS_PAD), jnp.float32),
    scratch_shapes=[
        pltpu.VMEM((D, ROWS_PAD), jnp.float32),
        pltpu.VMEM((D, ROWS_PAD), jnp.float32),
        pltpu.VMEM((D, 128), jnp.float32),
        pltpu.VMEM((D, 128), jnp.float32),
        pltpu.VMEM((D, 128), jnp.float32),
        pltpu.VMEM((D, 128), jnp.float32),
    ],
)


def _bcastD(v):
    return jnp.broadcast_to(v[:, None], (D, 128))


def _layer(ht, aggt, W, b, eps, gi, bi, go, bo):
    epsb = jnp.reshape(1.0 + eps, (1, 1))
    return _layer_call(epsb, ht, aggt, W.T, _bcastD(b),
                       _bcastD(gi), _bcastD(bi), _bcastD(go), _bcastD(bo))


def kernel(x, edge_index, W0, b0, W1, b1, eps0, eps1,
           g_in0, b_in0, g_out0, b_out0, g_in1, b_in1, g_out1, b_out1):
    src = edge_index[0].astype(jnp.int32)
    dst = edge_index[1].astype(jnp.int32)
    zeros = jnp.zeros((CPT * ROWS_PAD,), jnp.float32)

    xt = jnp.pad(x.T, ((0, 0), (0, ROWS_PAD - N)))
    agg0 = _sc_agg(xt.reshape(NW, CPT, ROWS_PAD), src, dst, zeros)
    h1t = _layer(xt, agg0.reshape(D, ROWS_PAD), W0, b0, eps0,
                 g_in0, b_in0, g_out0, b_out0)
    agg1 = _sc_agg(h1t.reshape(NW, CPT, ROWS_PAD), src, dst, zeros)
    h2t = _layer(h1t, agg1.reshape(D, ROWS_PAD), W1, b1, eps1,
                 g_in1, b_in1, g_out1, b_out1)
    return jnp.concatenate([h1t[:, :N].T, h2t[:, :N].T], axis=1)
